# baseline (speedup 1.0000x reference)
import functools

import jax
import jax.numpy as jnp
from jax import lax
from jax.experimental import pallas as pl
from jax.experimental.pallas import tpu as pltpu
from jax.experimental.pallas import tpu_sc as plsc

BATCH = 16384
EMBED_DIM = 64
NUM_CORES = 2
NUM_SUBCORES = 16
NUM_WORKERS = NUM_CORES * NUM_SUBCORES
NUM_TJ = 7813
TJ_PER_W = 245
G = 4
NUM_GROUPS = 62
LANES = 16
NUM_ID_VREGS = BATCH // LANES
ROW_CAP = 128
FLUSH_AT = ROW_CAP - LANES


def _pass(ids_h, t_h, out_h, idsall, hids, hslots, rowsflat, rowslots,
          bufs, bsems, osem, lo):
    iota = lax.broadcasted_iota(jnp.int32, (LANES,), 0)
    hi = lo + TJ_PER_W

    pltpu.sync_copy(ids_h, idsall)

    def p1(g, nh):
        v = idsall[pl.ds(g * LANES, LANES)]
        slotv = g * LANES + iota
        tjv = lax.shift_right_logical(v, 7)
        m = jnp.logical_and(tjv >= lo, tjv < hi)
        mcum = plsc.cumsum(m.astype(jnp.int32))
        pos = nh + mcum - 1
        plsc.store_scatter(hids, [pos], v, mask=m)
        plsc.store_scatter(hslots, [pos], slotv, mask=m)
        return nh + mcum[15]

    nh = lax.fori_loop(0, NUM_ID_VREGS, p1, jnp.int32(0))
    nhv = lax.shift_right_logical(nh + LANES - 1, 4)

    def flush(off2):
        nrv = lax.shift_right_logical(off2 + LANES - 1, 4)

        def fire_rows(rv, carry):
            slotv = rowslots[pl.ds(rv * LANES, LANES)]
            for k in range(LANES):
                @pl.when(rv * LANES + k < off2)
                def _(k=k, rv=rv, slotv=slotv):
                    slotk = slotv[k]
                    pltpu.make_async_copy(
                        rowsflat.at[pl.ds((rv * LANES + k) * EMBED_DIM,
                                          EMBED_DIM)],
                        out_h.at[pl.ds(slotk * EMBED_DIM, EMBED_DIM)],
                        osem).start()
            return carry

        lax.fori_loop(0, nrv, fire_rows, 0)

        def drain_rows(r, _):
            pltpu.make_async_copy(
                out_h.at[pl.ds(0, EMBED_DIM)],
                rowsflat.at[pl.ds(0, EMBED_DIM)], osem).wait()
            return _

        lax.fori_loop(0, off2, drain_rows, 0)

    def fire_group(t, p):
        t0c = jnp.minimum(lo + t * G, NUM_TJ - G)
        pltpu.make_async_copy(
            t_h.at[:, pl.ds(t0c * 128, G * 128)], bufs.at[p],
            bsems[p]).start()

    def body(it, off2):
        for p in range(2):
            t = it * 2 + p
            valid = t >= 2
            tp = jnp.maximum(t - 2, 0)
            t0c = jnp.minimum(lo + tp * G, NUM_TJ - G)

            @pl.when(valid)
            def _(p=p):
                pltpu.make_async_copy(
                    t_h.at[pl.ds(0, EMBED_DIM), pl.ds(0, G * 128)],
                    bufs.at[p], bsems[p]).wait()

            def scanv(hv, off2, p=p, valid=valid, t0c=t0c):
                lm = (hv * LANES + iota) < nh
                hvv = hids[pl.ds(hv * LANES, LANES)]
                tjv = lax.shift_right_logical(hvv, 7)
                mt = jnp.logical_and(
                    jnp.logical_and(lm, valid),
                    jnp.logical_and(tjv >= t0c, tjv < t0c + G))
                mcum = plsc.cumsum(mt.astype(jnp.int32))
                cnt = mcum[15]

                do_flush = jnp.logical_and(cnt > 0, off2 >= FLUSH_AT)

                @pl.when(do_flush)
                def _():
                    flush(off2)

                off2 = jnp.where(do_flush, 0, off2)

                @pl.when(cnt > 0)
                def _(off2=off2):
                    slotv = hslots[pl.ds(hv * LANES, LANES)]
                    colv = hvv - t0c * 128
                    posv = off2 + mcum - 1
                    for d in range(EMBED_DIM):
                        gv = plsc.load_gather(
                            bufs.at[p],
                            [jnp.full((LANES,), jnp.int32(d), jnp.int32),
                             colv],
                            mask=mt)
                        plsc.store_scatter(
                            rowsflat, [posv * EMBED_DIM + d], gv, mask=mt)
                    plsc.store_scatter(rowslots, [posv], slotv, mask=mt)

                return off2 + cnt

            off2 = lax.fori_loop(0, nhv, scanv, off2)

            @pl.when(t <= NUM_GROUPS - 1)
            def _(p=p, t=t):
                fire_group(t, p)
        return off2

    off2 = lax.fori_loop(0, NUM_GROUPS // 2 + 1, body, jnp.int32(0))

    @pl.when(off2 > 0)
    def _():
        flush(off2)


def _lookup_body(user_ids_h, item_ids_h, user_t_h, item_t_h,
                 user_out_h, item_out_h,
                 idsall, hids, hslots, rowsflat, rowslots, bufs,
                 bsem0, bsem1, osem):
    c = lax.axis_index("c")
    s = lax.axis_index("s")
    wid = s * NUM_CORES + c
    lo = wid * TJ_PER_W
    bsems = (bsem0, bsem1)
    _pass(user_ids_h, user_t_h, user_out_h, idsall, hids, hslots,
          rowsflat, rowslots, bufs, bsems, osem, lo)
    _pass(item_ids_h, item_t_h, item_out_h, idsall, hids, hslots,
          rowsflat, rowslots, bufs, bsems, osem, lo)


@jax.jit
def _lookup(user_ids, item_ids, user_t, item_t):
    mesh = plsc.VectorSubcoreMesh(core_axis_name="c", subcore_axis_name="s")
    run = functools.partial(
        pl.kernel,
        mesh=mesh,
        out_type=[
            jax.ShapeDtypeStruct((BATCH * EMBED_DIM,), jnp.float32),
            jax.ShapeDtypeStruct((BATCH * EMBED_DIM,), jnp.float32),
        ],
        scratch_types=[
            pltpu.VMEM((BATCH,), jnp.int32),
            pltpu.VMEM((BATCH,), jnp.int32),
            pltpu.VMEM((BATCH,), jnp.int32),
            pltpu.VMEM((ROW_CAP * EMBED_DIM,), jnp.float32),
            pltpu.VMEM((ROW_CAP,), jnp.int32),
            pltpu.VMEM((2, EMBED_DIM, G * 128), jnp.float32),
            pltpu.SemaphoreType.DMA,
            pltpu.SemaphoreType.DMA,
            pltpu.SemaphoreType.DMA,
        ],
        compiler_params=pltpu.CompilerParams(
            use_tc_tiling_on_sc=True, needs_layout_passes=False),
    )(_lookup_body)
    return run(user_ids, item_ids, user_t, item_t)


def kernel(user_ids, item_ids, user_table, item_table):
    user_flat, item_flat = _lookup(
        user_ids, item_ids, user_table.T, item_table.T)
    return (user_flat.reshape(BATCH, EMBED_DIM),
            item_flat.reshape(BATCH, EMBED_DIM))


# speedup vs baseline: 1.0087x; 1.0087x over previous
import functools

import jax
import jax.numpy as jnp
from jax import lax
from jax.experimental import pallas as pl
from jax.experimental.pallas import tpu as pltpu
from jax.experimental.pallas import tpu_sc as plsc

BATCH = 16384
EMBED_DIM = 64
NUM_CORES = 2
NUM_SUBCORES = 16
NUM_WORKERS = NUM_CORES * NUM_SUBCORES
NUM_TJ = 7813
TJ_PER_W = 245
G = 4
NUM_GROUPS = 62
LANES = 16
NUM_ID_VREGS = BATCH // LANES
ROW_CAP = 128
FLUSH_AT = ROW_CAP - LANES


def _pass(ids_h, t_h, out_h, idsall, hids, hslots, rowsflat, rowslots,
          bufs, bsems, osem, lo):
    iota = lax.broadcasted_iota(jnp.int32, (LANES,), 0)
    hi = lo + TJ_PER_W

    pltpu.sync_copy(ids_h, idsall)

    def p1(g, nh):
        v = idsall[pl.ds(g * LANES, LANES)]
        slotv = g * LANES + iota
        tjv = lax.shift_right_logical(v, 7)
        m = jnp.logical_and(tjv >= lo, tjv < hi)
        mi = m.astype(jnp.int32)
        pos = nh + plsc.cumsum(mi) - 1
        plsc.store_scatter(hids, [pos], v, mask=m)
        plsc.store_scatter(hslots, [pos], slotv, mask=m)
        return nh + plsc.all_reduce_population_count(m)[0]

    nh = lax.fori_loop(0, NUM_ID_VREGS, p1, jnp.int32(0))
    nhv = lax.shift_right_logical(nh + LANES - 1, 4)

    def flush(off2):
        nrv = lax.shift_right_logical(off2 + LANES - 1, 4)

        def fire_rows(rv, carry):
            slotv = rowslots[pl.ds(rv * LANES, LANES)]
            for k in range(LANES):
                @pl.when(rv * LANES + k < off2)
                def _(k=k, rv=rv, slotv=slotv):
                    slotk = slotv[k]
                    pltpu.make_async_copy(
                        rowsflat.at[pl.ds((rv * LANES + k) * EMBED_DIM,
                                          EMBED_DIM)],
                        out_h.at[pl.ds(slotk * EMBED_DIM, EMBED_DIM)],
                        osem).start()
            return carry

        lax.fori_loop(0, nrv, fire_rows, 0)

        def drain_rows(r, _):
            pltpu.make_async_copy(
                out_h.at[pl.ds(0, EMBED_DIM)],
                rowsflat.at[pl.ds(0, EMBED_DIM)], osem).wait()
            return _

        lax.fori_loop(0, off2, drain_rows, 0)

    def fire_group(t, p):
        t0c = jnp.minimum(lo + t * G, NUM_TJ - G)
        pltpu.make_async_copy(
            t_h.at[:, pl.ds(t0c * 128, G * 128)], bufs.at[p],
            bsems[p]).start()

    def body(it, off2):
        for p in range(2):
            t = it * 2 + p
            valid = t >= 2
            tp = jnp.maximum(t - 2, 0)
            t0c = jnp.minimum(lo + tp * G, NUM_TJ - G)

            @pl.when(valid)
            def _(p=p):
                pltpu.make_async_copy(
                    t_h.at[pl.ds(0, EMBED_DIM), pl.ds(0, G * 128)],
                    bufs.at[p], bsems[p]).wait()

            def scanv(hv, off2, p=p, valid=valid, t0c=t0c):
                lm = (hv * LANES + iota) < nh
                hvv = hids[pl.ds(hv * LANES, LANES)]
                slotv = hslots[pl.ds(hv * LANES, LANES)]
                tjv = lax.shift_right_logical(hvv, 7)
                mt = jnp.logical_and(
                    jnp.logical_and(lm, valid),
                    jnp.logical_and(tjv >= t0c, tjv < t0c + G))
                cnt = plsc.all_reduce_population_count(mt)[0]

                do_flush = jnp.logical_and(cnt > 0, off2 >= FLUSH_AT)

                @pl.when(do_flush)
                def _():
                    flush(off2)

                off2 = jnp.where(do_flush, 0, off2)

                @pl.when(cnt > 0)
                def _(off2=off2):
                    colv = hvv - t0c * 128
                    posv = off2 + plsc.cumsum(mt.astype(jnp.int32)) - 1
                    for d in range(EMBED_DIM):
                        gv = plsc.load_gather(
                            bufs.at[p],
                            [jnp.full((LANES,), jnp.int32(d), jnp.int32),
                             colv],
                            mask=mt)
                        plsc.store_scatter(
                            rowsflat, [posv * EMBED_DIM + d], gv, mask=mt)
                    plsc.store_scatter(rowslots, [posv], slotv, mask=mt)

                return off2 + cnt

            off2 = lax.fori_loop(0, nhv, scanv, off2)

            @pl.when(t <= NUM_GROUPS - 1)
            def _(p=p, t=t):
                fire_group(t, p)
        return off2

    off2 = lax.fori_loop(0, NUM_GROUPS // 2 + 1, body, jnp.int32(0))

    @pl.when(off2 > 0)
    def _():
        flush(off2)


def _lookup_body(user_ids_h, item_ids_h, user_t_h, item_t_h,
                 user_out_h, item_out_h,
                 idsall, hids, hslots, rowsflat, rowslots, bufs,
                 bsem0, bsem1, osem):
    c = lax.axis_index("c")
    s = lax.axis_index("s")
    wid = s * NUM_CORES + c
    lo = wid * TJ_PER_W
    bsems = (bsem0, bsem1)
    _pass(user_ids_h, user_t_h, user_out_h, idsall, hids, hslots,
          rowsflat, rowslots, bufs, bsems, osem, lo)
    _pass(item_ids_h, item_t_h, item_out_h, idsall, hids, hslots,
          rowsflat, rowslots, bufs, bsems, osem, lo)


@jax.jit
def _lookup(user_ids, item_ids, user_t, item_t):
    mesh = plsc.VectorSubcoreMesh(core_axis_name="c", subcore_axis_name="s")
    run = functools.partial(
        pl.kernel,
        mesh=mesh,
        out_type=[
            jax.ShapeDtypeStruct((BATCH * EMBED_DIM,), jnp.float32),
            jax.ShapeDtypeStruct((BATCH * EMBED_DIM,), jnp.float32),
        ],
        scratch_types=[
            pltpu.VMEM((BATCH,), jnp.int32),
            pltpu.VMEM((BATCH,), jnp.int32),
            pltpu.VMEM((BATCH,), jnp.int32),
            pltpu.VMEM((ROW_CAP * EMBED_DIM,), jnp.float32),
            pltpu.VMEM((ROW_CAP,), jnp.int32),
            pltpu.VMEM((2, EMBED_DIM, G * 128), jnp.float32),
            pltpu.SemaphoreType.DMA,
            pltpu.SemaphoreType.DMA,
            pltpu.SemaphoreType.DMA,
        ],
        compiler_params=pltpu.CompilerParams(
            use_tc_tiling_on_sc=True, needs_layout_passes=False),
    )(_lookup_body)
    return run(user_ids, item_ids, user_t, item_t)


def kernel(user_ids, item_ids, user_table, item_table):
    user_flat, item_flat = _lookup(
        user_ids, item_ids, user_table.T, item_table.T)
    return (user_flat.reshape(BATCH, EMBED_DIM),
            item_flat.reshape(BATCH, EMBED_DIM))


# speedup vs baseline: 1.4590x; 1.4464x over previous
import functools

import jax
import jax.numpy as jnp
from jax import lax
from jax.experimental import pallas as pl
from jax.experimental.pallas import tpu as pltpu
from jax.experimental.pallas import tpu_sc as plsc

BATCH = 16384
EMBED_DIM = 64
NUM_CORES = 2
NUM_SUBCORES = 16
NUM_WORKERS = NUM_CORES * NUM_SUBCORES
NUM_TJ = 7813
TJ_PER_W = 245
G = 4
NUM_GROUPS = 62
LANES = 16
NUM_ID_VREGS = BATCH // LANES
ROW_CAP = 128
IDS_CHUNK = 4096
FLUSH_AT = ROW_CAP - LANES


def _pass(ids_h, t_h, out_h, idsall, hids, hslots, rowsflat, rowslots,
          grouplist, bufs, bsems, osem, lo):
    iota = lax.broadcasted_iota(jnp.int32, (LANES,), 0)
    hi = lo + TJ_PER_W

    nh = jnp.int32(0)
    for chunk in range(BATCH // IDS_CHUNK):
        pltpu.sync_copy(ids_h.at[pl.ds(chunk * IDS_CHUNK, IDS_CHUNK)],
                        idsall)

        def p1(g, nh, chunk=chunk):
            v = idsall[pl.ds(g * LANES, LANES)]
            slotv = chunk * IDS_CHUNK + g * LANES + iota
            tjv = lax.shift_right_logical(v, 7)
            m = jnp.logical_and(tjv >= lo, tjv < hi)
            mi = m.astype(jnp.int32)
            pos = nh + plsc.cumsum(mi) - 1
            plsc.store_scatter(hids, [pos], v, mask=m)
            plsc.store_scatter(hslots, [pos], slotv, mask=m)
            return nh + plsc.all_reduce_population_count(m)[0]

        nh = lax.fori_loop(0, IDS_CHUNK // LANES, p1, nh)
    nhv = lax.shift_right_logical(nh + LANES - 1, 4)

    def flush(off2):
        nrv = lax.shift_right_logical(off2 + LANES - 1, 4)

        def fire_rows(rv, carry):
            slotv = rowslots[pl.ds(rv * LANES, LANES)]
            for k in range(LANES):
                @pl.when(rv * LANES + k < off2)
                def _(k=k, rv=rv, slotv=slotv):
                    slotk = slotv[k]
                    pltpu.make_async_copy(
                        rowsflat.at[pl.ds((rv * LANES + k) * EMBED_DIM,
                                          EMBED_DIM)],
                        out_h.at[pl.ds(slotk * EMBED_DIM, EMBED_DIM)],
                        osem).start()
            return carry

        lax.fori_loop(0, nrv, fire_rows, 0)

        def drain_rows(r, _):
            pltpu.make_async_copy(
                out_h.at[pl.ds(0, EMBED_DIM)],
                rowsflat.at[pl.ds(0, EMBED_DIM)], osem).wait()
            return _

        lax.fori_loop(0, off2, drain_rows, 0)

    def fire_group(t, p):
        t0c = jnp.minimum(lo + t * G, NUM_TJ - G)
        pltpu.make_async_copy(
            t_h.at[:, pl.ds(t0c * 128, G * 128)], bufs.at[p],
            bsems[p]).start()

    def body(it, off2):
        for p in range(2):
            t = it * 2 + p
            valid = t >= 2
            tp = jnp.maximum(t - 2, 0)
            t0c = jnp.minimum(lo + tp * G, NUM_TJ - G)

            @pl.when(valid)
            def _(p=p):
                pltpu.make_async_copy(
                    t_h.at[pl.ds(0, EMBED_DIM), pl.ds(0, G * 128)],
                    bufs.at[p], bsems[p]).wait()

            def scanv(hv, goff, p=p, valid=valid, t0c=t0c):
                lm = (hv * LANES + iota) < nh
                hvv = hids[pl.ds(hv * LANES, LANES)]
                tjv = lax.shift_right_logical(hvv, 7)
                mt = jnp.logical_and(
                    jnp.logical_and(lm, valid),
                    jnp.logical_and(tjv >= t0c, tjv < t0c + G))
                cnt = plsc.all_reduce_population_count(mt)[0]

                @pl.when(cnt > 0)
                def _(goff=goff):
                    slotv = hslots[pl.ds(hv * LANES, LANES)]
                    colv = hvv - t0c * 128
                    posv = goff + plsc.cumsum(mt.astype(jnp.int32)) - 1
                    plsc.store_scatter(
                        grouplist, [posv], slotv * 512 + colv, mask=mt)

                return goff + cnt

            goff = lax.fori_loop(0, nhv, scanv, jnp.int32(0))

            def extractv(ev, off2, p=p):
                lm = (ev * LANES + iota) < goff
                pk = grouplist[pl.ds(ev * LANES, LANES)]
                slotv = lax.shift_right_logical(pk, 9)
                colv = jnp.bitwise_and(pk, jnp.int32(511))

                do_flush = off2 >= FLUSH_AT

                @pl.when(do_flush)
                def _():
                    flush(off2)

                off2 = jnp.where(do_flush, 0, off2)
                posv = off2 + plsc.cumsum(lm.astype(jnp.int32)) - 1
                for d in range(EMBED_DIM):
                    gv = plsc.load_gather(
                        bufs.at[p],
                        [jnp.full((LANES,), jnp.int32(d), jnp.int32),
                         colv],
                        mask=lm)
                    plsc.store_scatter(
                        rowsflat, [posv * EMBED_DIM + d], gv, mask=lm)
                plsc.store_scatter(rowslots, [posv], slotv, mask=lm)
                return off2 + plsc.all_reduce_population_count(lm)[0]

            negv = lax.shift_right_logical(goff + LANES - 1, 4)
            off2 = lax.fori_loop(0, negv, extractv, off2)

            @pl.when(t <= NUM_GROUPS - 1)
            def _(p=p, t=t):
                fire_group(t, p)
        return off2

    off2 = lax.fori_loop(0, NUM_GROUPS // 2 + 1, body, jnp.int32(0))

    @pl.when(off2 > 0)
    def _():
        flush(off2)


def _lookup_body(user_ids_h, item_ids_h, user_t_h, item_t_h,
                 user_out_h, item_out_h,
                 idsall, hids, hslots, rowsflat, rowslots, grouplist, bufs,
                 bsem0, bsem1, osem):
    c = lax.axis_index("c")
    s = lax.axis_index("s")
    wid = s * NUM_CORES + c
    lo = wid * TJ_PER_W
    bsems = (bsem0, bsem1)
    _pass(user_ids_h, user_t_h, user_out_h, idsall, hids, hslots,
          rowsflat, rowslots, grouplist, bufs, bsems, osem, lo)
    _pass(item_ids_h, item_t_h, item_out_h, idsall, hids, hslots,
          rowsflat, rowslots, grouplist, bufs, bsems, osem, lo)


@jax.jit
def _lookup(user_ids, item_ids, user_t, item_t):
    mesh = plsc.VectorSubcoreMesh(core_axis_name="c", subcore_axis_name="s")
    run = functools.partial(
        pl.kernel,
        mesh=mesh,
        out_type=[
            jax.ShapeDtypeStruct((BATCH * EMBED_DIM,), jnp.float32),
            jax.ShapeDtypeStruct((BATCH * EMBED_DIM,), jnp.float32),
        ],
        scratch_types=[
            pltpu.VMEM((IDS_CHUNK,), jnp.int32),
            pltpu.VMEM((BATCH,), jnp.int32),
            pltpu.VMEM((BATCH,), jnp.int32),
            pltpu.VMEM((ROW_CAP * EMBED_DIM,), jnp.float32),
            pltpu.VMEM((ROW_CAP,), jnp.int32),
            pltpu.VMEM((BATCH,), jnp.int32),
            pltpu.VMEM((2, EMBED_DIM, G * 128), jnp.float32),
            pltpu.SemaphoreType.DMA,
            pltpu.SemaphoreType.DMA,
            pltpu.SemaphoreType.DMA,
        ],
        compiler_params=pltpu.CompilerParams(
            use_tc_tiling_on_sc=True, needs_layout_passes=False),
    )(_lookup_body)
    return run(user_ids, item_ids, user_t, item_t)


def kernel(user_ids, item_ids, user_table, item_table):
    user_flat, item_flat = _lookup(
        user_ids, item_ids, user_table.T, item_table.T)
    return (user_flat.reshape(BATCH, EMBED_DIM),
            item_flat.reshape(BATCH, EMBED_DIM))


# speedup vs baseline: 1.4612x; 1.0015x over previous
import functools

import jax
import jax.numpy as jnp
from jax import lax
from jax.experimental import pallas as pl
from jax.experimental.pallas import tpu as pltpu
from jax.experimental.pallas import tpu_sc as plsc

BATCH = 16384
EMBED_DIM = 64
NUM_CORES = 2
NUM_SUBCORES = 16
NUM_WORKERS = NUM_CORES * NUM_SUBCORES
NUM_TJ = 7813
TJ_PER_W = 245
G = 4
NUM_GROUPS = 62
LANES = 16
NUM_ID_VREGS = BATCH // LANES
ROW_CAP = 128
IDS_CHUNK = 4096
FLUSH_AT = ROW_CAP - LANES


def _pass(ids_h, t_h, out_h, idsall, hids, hslots, rowsflat, rowslots,
          grouplist, bufs, bsems, osem, lo):
    iota = lax.broadcasted_iota(jnp.int32, (LANES,), 0)
    hi = lo + TJ_PER_W

    nh = jnp.int32(0)
    for chunk in range(BATCH // IDS_CHUNK):
        pltpu.sync_copy(ids_h.at[pl.ds(chunk * IDS_CHUNK, IDS_CHUNK)],
                        idsall)

        def p1(g, nh, chunk=chunk):
            v = idsall[pl.ds(g * LANES, LANES)]
            slotv = chunk * IDS_CHUNK + g * LANES + iota
            tjv = lax.shift_right_logical(v, 7)
            m = jnp.logical_and(tjv >= lo, tjv < hi)
            mi = m.astype(jnp.int32)
            pos = nh + plsc.cumsum(mi) - 1
            plsc.store_scatter(hids, [pos], v, mask=m)
            plsc.store_scatter(hslots, [pos], slotv, mask=m)
            return nh + plsc.all_reduce_population_count(m)[0]

        nh = lax.fori_loop(0, IDS_CHUNK // LANES, p1, nh)
    nhv = lax.shift_right_logical(nh + LANES - 1, 4)

    def flush(off2):
        nrv = lax.shift_right_logical(off2 + LANES - 1, 4)

        def fire_rows(rv, carry):
            slotv = rowslots[pl.ds(rv * LANES, LANES)]
            for k in range(LANES):
                @pl.when(rv * LANES + k < off2)
                def _(k=k, rv=rv, slotv=slotv):
                    slotk = slotv[k]
                    pltpu.make_async_copy(
                        rowsflat.at[pl.ds((rv * LANES + k) * EMBED_DIM,
                                          EMBED_DIM)],
                        out_h.at[pl.ds(slotk * EMBED_DIM, EMBED_DIM)],
                        osem).start()
            return carry

        lax.fori_loop(0, nrv, fire_rows, 0)

        def drain_rows(r, _):
            pltpu.make_async_copy(
                out_h.at[pl.ds(0, EMBED_DIM)],
                rowsflat.at[pl.ds(0, EMBED_DIM)], osem).wait()
            return _

        lax.fori_loop(0, off2, drain_rows, 0)

    def fire_group(t, p):
        t0c = jnp.minimum(lo + t * G, NUM_TJ - G)
        pltpu.make_async_copy(
            t_h.at[:, pl.ds(t0c * 128, G * 128)], bufs.at[p],
            bsems[p]).start()

    def body(it, off2):
        for p in range(2):
            t = it * 2 + p
            valid = t >= 2
            tp = jnp.maximum(t - 2, 0)
            t0c = jnp.minimum(lo + tp * G, NUM_TJ - G)

            @pl.when(valid)
            def _(p=p):
                pltpu.make_async_copy(
                    t_h.at[pl.ds(0, EMBED_DIM), pl.ds(0, G * 128)],
                    bufs.at[p], bsems[p]).wait()

            def scanv(hv, goff, p=p, valid=valid, t0c=t0c):
                lm = (hv * LANES + iota) < nh
                hvv = hids[pl.ds(hv * LANES, LANES)]
                tjv = lax.shift_right_logical(hvv, 7)
                mt = jnp.logical_and(
                    jnp.logical_and(lm, valid),
                    jnp.logical_and(tjv >= t0c, tjv < t0c + G))
                mcum = plsc.cumsum(mt.astype(jnp.int32))
                cnt = mcum[15]

                @pl.when(cnt > 0)
                def _(goff=goff):
                    slotv = hslots[pl.ds(hv * LANES, LANES)]
                    colv = hvv - t0c * 128
                    posv = goff + mcum - 1
                    plsc.store_scatter(
                        grouplist, [posv], slotv * 512 + colv, mask=mt)

                return goff + cnt

            goff = lax.fori_loop(0, nhv, scanv, jnp.int32(0))

            def extractv(ev, off2, p=p):
                lm = (ev * LANES + iota) < goff
                pk = grouplist[pl.ds(ev * LANES, LANES)]
                slotv = lax.shift_right_logical(pk, 9)
                colv = jnp.bitwise_and(pk, jnp.int32(511))

                do_flush = off2 >= FLUSH_AT

                @pl.when(do_flush)
                def _():
                    flush(off2)

                off2 = jnp.where(do_flush, 0, off2)
                posv = off2 + plsc.cumsum(lm.astype(jnp.int32)) - 1
                for d in range(EMBED_DIM):
                    gv = plsc.load_gather(
                        bufs.at[p],
                        [jnp.full((LANES,), jnp.int32(d), jnp.int32),
                         colv],
                        mask=lm)
                    plsc.store_scatter(
                        rowsflat, [posv * EMBED_DIM + d], gv, mask=lm)
                plsc.store_scatter(rowslots, [posv], slotv, mask=lm)
                return off2 + plsc.all_reduce_population_count(lm)[0]

            negv = lax.shift_right_logical(goff + LANES - 1, 4)
            off2 = lax.fori_loop(0, negv, extractv, off2)

            @pl.when(t <= NUM_GROUPS - 1)
            def _(p=p, t=t):
                fire_group(t, p)
        return off2

    off2 = lax.fori_loop(0, NUM_GROUPS // 2 + 1, body, jnp.int32(0))

    @pl.when(off2 > 0)
    def _():
        flush(off2)


def _lookup_body(user_ids_h, item_ids_h, user_t_h, item_t_h,
                 user_out_h, item_out_h,
                 idsall, hids, hslots, rowsflat, rowslots, grouplist, bufs,
                 bsem0, bsem1, osem):
    c = lax.axis_index("c")
    s = lax.axis_index("s")
    wid = s * NUM_CORES + c
    lo = wid * TJ_PER_W
    bsems = (bsem0, bsem1)
    _pass(user_ids_h, user_t_h, user_out_h, idsall, hids, hslots,
          rowsflat, rowslots, grouplist, bufs, bsems, osem, lo)
    _pass(item_ids_h, item_t_h, item_out_h, idsall, hids, hslots,
          rowsflat, rowslots, grouplist, bufs, bsems, osem, lo)


@jax.jit
def _lookup(user_ids, item_ids, user_t, item_t):
    mesh = plsc.VectorSubcoreMesh(core_axis_name="c", subcore_axis_name="s")
    run = functools.partial(
        pl.kernel,
        mesh=mesh,
        out_type=[
            jax.ShapeDtypeStruct((BATCH * EMBED_DIM,), jnp.float32),
            jax.ShapeDtypeStruct((BATCH * EMBED_DIM,), jnp.float32),
        ],
        scratch_types=[
            pltpu.VMEM((IDS_CHUNK,), jnp.int32),
            pltpu.VMEM((BATCH,), jnp.int32),
            pltpu.VMEM((BATCH,), jnp.int32),
            pltpu.VMEM((ROW_CAP * EMBED_DIM,), jnp.float32),
            pltpu.VMEM((ROW_CAP,), jnp.int32),
            pltpu.VMEM((BATCH,), jnp.int32),
            pltpu.VMEM((2, EMBED_DIM, G * 128), jnp.float32),
            pltpu.SemaphoreType.DMA,
            pltpu.SemaphoreType.DMA,
            pltpu.SemaphoreType.DMA,
        ],
        compiler_params=pltpu.CompilerParams(
            use_tc_tiling_on_sc=True, needs_layout_passes=False),
    )(_lookup_body)
    return run(user_ids, item_ids, user_t, item_t)


def kernel(user_ids, item_ids, user_table, item_table):
    user_flat, item_flat = _lookup(
        user_ids, item_ids, user_table.T, item_table.T)
    return (user_flat.reshape(BATCH, EMBED_DIM),
            item_flat.reshape(BATCH, EMBED_DIM))
